# baseline (device time: 179893 ns/iter reference)
import jax
import jax.numpy as jnp
from jax import lax
from jax.experimental import pallas as pl
from jax.experimental.pallas import tpu as pltpu

B = 32
H = 16
D = 128
BS = 32
NB = 256
QUAD = 4
SCALE = D ** -0.5


def kernel(Q, K, V, bt, lens):
    n_pages = K.shape[0]
    my_pages = n_pages // QUAD
    nkeys = my_pages * BS
    lens2 = lens.reshape(B, 1)

    x = lax.axis_index("x")
    z = lax.axis_index("z")
    quad_idx = (x * 2 + z).astype(jnp.int32).reshape(1)

    Q2 = Q.reshape(B, H * D)
    K2 = K.reshape(n_pages, BS, H * D)
    V2 = V.reshape(n_pages, BS, H * D)

    def body(qi_ref, bt_ref, lens_ref, q_ref, k_ref, v_ref, out_ref,
             acc_o, acc_l, counts_all, send_o, recv_o, recv_l,
             send_sems, recv_sems):
        h = pl.program_id(0)
        my_x = lax.axis_index("x")
        my_y = lax.axis_index("y")
        my_z = lax.axis_index("z")
        partners = [
            (my_x, 1 - my_y, my_z),
            (1 - my_x, my_y, my_z),
            (my_x, my_y, 1 - my_z),
        ]

        @pl.when(h == 0)
        def _init():
            barrier_sem = pltpu.get_barrier_semaphore()
            for p in partners:
                pl.semaphore_signal(
                    barrier_sem, inc=1, device_id=p,
                    device_id_type=pl.DeviceIdType.MESH,
                )
            pl.semaphore_wait(barrier_sem, 3)
            page_ids = (my_y * n_pages + qi_ref[0] * my_pages
                        + lax.broadcasted_iota(jnp.int32, (1, my_pages, 1), 1))
            btv = bt_ref[...]
            col = lax.broadcasted_iota(jnp.int32, (1, 1, NB), 2)
            hit = (btv[:, None, :] == page_ids) & (
                col < lens_ref[...][:, :, None])
            counts_all[...] = jnp.sum(hit.astype(jnp.float32), axis=2)

        counts_k = jnp.broadcast_to(
            counts_all[...][:, :, None], (B, my_pages, BS)).reshape(B, nkeys)

        q = (q_ref[...] * SCALE).astype(jnp.bfloat16)
        k = k_ref[...].reshape(nkeys, D).astype(jnp.bfloat16)
        s = lax.dot_general(
            q, k, (((1,), (1,)), ((), ())),
            preferred_element_type=jnp.float32)
        e = jnp.exp(s) * counts_k
        acc_l[h, :] = jnp.sum(e, axis=1)
        v = v_ref[...].reshape(nkeys, D).astype(jnp.bfloat16)
        acc_o[h] = lax.dot_general(
            e.astype(jnp.bfloat16), v, (((1,), (0,)), ((), ())),
            preferred_element_type=jnp.float32)

        @pl.when(h == H - 1)
        def _finish():
            for st, p in enumerate(partners):
                send_o[...] = acc_o[...].astype(jnp.bfloat16)
                rdma_o = pltpu.make_async_remote_copy(
                    src_ref=send_o, dst_ref=recv_o.at[st],
                    send_sem=send_sems.at[st, 0], recv_sem=recv_sems.at[st, 0],
                    device_id=p, device_id_type=pl.DeviceIdType.MESH,
                )
                rdma_l = pltpu.make_async_remote_copy(
                    src_ref=acc_l, dst_ref=recv_l.at[st],
                    send_sem=send_sems.at[st, 1], recv_sem=recv_sems.at[st, 1],
                    device_id=p, device_id_type=pl.DeviceIdType.MESH,
                )
                rdma_o.start()
                rdma_l.start()
                rdma_o.wait()
                rdma_l.wait()
                acc_o[...] += recv_o[st].astype(jnp.float32)
                acc_l[...] += recv_l[st]
            res = acc_o[...] / acc_l[...][:, :, None]
            out_ref[...] = jnp.transpose(res, (1, 0, 2)).reshape(B, 1, H, D)

    grid_spec = pltpu.PrefetchScalarGridSpec(
        num_scalar_prefetch=1,
        grid=(H,),
        in_specs=[
            pl.BlockSpec((B, NB), lambda h, qi: (0, 0)),
            pl.BlockSpec((B, 1), lambda h, qi: (0, 0)),
            pl.BlockSpec((B, D), lambda h, qi: (0, h)),
            pl.BlockSpec((my_pages, BS, D), lambda h, qi: (qi[0], 0, h)),
            pl.BlockSpec((my_pages, BS, D), lambda h, qi: (qi[0], 0, h)),
        ],
        out_specs=pl.BlockSpec((B, 1, H, D), lambda h, qi: (0, 0, 0, 0)),
        scratch_shapes=[
            pltpu.VMEM((H, B, D), jnp.float32),
            pltpu.VMEM((H, B), jnp.float32),
            pltpu.VMEM((B, my_pages), jnp.float32),
            pltpu.VMEM((H, B, D), jnp.bfloat16),
            pltpu.VMEM((3, H, B, D), jnp.bfloat16),
            pltpu.VMEM((3, H, B), jnp.float32),
            pltpu.SemaphoreType.DMA((3, 2)),
            pltpu.SemaphoreType.DMA((3, 2)),
        ],
    )

    return pl.pallas_call(
        body,
        grid_spec=grid_spec,
        out_shape=jax.ShapeDtypeStruct((B, 1, H, D), jnp.float32),
        compiler_params=pltpu.CompilerParams(
            dimension_semantics=("arbitrary",),
            collective_id=0,
            vmem_limit_bytes=100 * 1024 * 1024,
        ),
    )(quad_idx, bt, lens2, Q2, K2, V2)


# device time: 173458 ns/iter; 1.0371x vs baseline; 1.0371x over previous
import jax
import jax.numpy as jnp
from jax import lax
from jax.experimental import pallas as pl
from jax.experimental.pallas import tpu as pltpu

B = 32
H = 16
D = 128
BS = 32
NB = 256
QUAD = 4
CHUNK = 16
SCALE = D ** -0.5


def kernel(Q, K, V, bt, lens):
    n_pages = K.shape[0]
    my_pages = n_pages // QUAD
    n_steps = my_pages // CHUNK
    nkeys = CHUNK * BS
    lens2 = lens.reshape(B, 1)

    x = lax.axis_index("x")
    z = lax.axis_index("z")
    chunk_base = ((x * 2 + z) * n_steps).astype(jnp.int32).reshape(1)

    Q2 = Q.reshape(B, H * D)
    K2 = K.reshape(n_pages, BS, H * D)
    V2 = V.reshape(n_pages, BS, H * D)

    def body(cb_ref, bt_ref, lens_ref, q_ref, k_ref, v_ref, out_ref,
             acc_o, acc_l, send_o, recv_o, recv_l, send_sems, recv_sems):
        j = pl.program_id(0)
        my_x = lax.axis_index("x")
        my_y = lax.axis_index("y")
        my_z = lax.axis_index("z")
        partners = [
            (my_x, 1 - my_y, my_z),
            (1 - my_x, my_y, my_z),
            (my_x, my_y, 1 - my_z),
        ]

        @pl.when(j == 0)
        def _init():
            acc_o[...] = jnp.zeros_like(acc_o)
            acc_l[...] = jnp.zeros_like(acc_l)
            barrier_sem = pltpu.get_barrier_semaphore()
            for p in partners:
                pl.semaphore_signal(
                    barrier_sem, inc=1, device_id=p,
                    device_id_type=pl.DeviceIdType.MESH,
                )
            pl.semaphore_wait(barrier_sem, 3)

        page_ids = (my_y * n_pages + (cb_ref[0] + j) * CHUNK
                    + lax.broadcasted_iota(jnp.int32, (1, CHUNK, 1), 1))
        btv = bt_ref[...]
        col = lax.broadcasted_iota(jnp.int32, (1, 1, NB), 2)
        hit = (btv[:, None, :] == page_ids) & (col < lens_ref[...][:, :, None])
        counts = jnp.sum(hit.astype(jnp.float32), axis=2)
        counts_k = jnp.broadcast_to(
            counts[:, :, None], (B, CHUNK, BS)).reshape(B, nkeys)

        qb = (q_ref[...] * SCALE).astype(jnp.bfloat16)
        kb = k_ref[...].reshape(nkeys, H * D).astype(jnp.bfloat16)
        vb = v_ref[...].reshape(nkeys, H * D).astype(jnp.bfloat16)
        for h in range(H):
            sl = slice(h * D, (h + 1) * D)
            s = lax.dot_general(
                qb[:, sl], kb[:, sl], (((1,), (1,)), ((), ())),
                preferred_element_type=jnp.float32)
            e = jnp.exp(s) * counts_k
            acc_l[h, :] += jnp.sum(e, axis=1)
            acc_o[h] += lax.dot_general(
                e.astype(jnp.bfloat16), vb[:, sl], (((1,), (0,)), ((), ())),
                preferred_element_type=jnp.float32)

        @pl.when(j == n_steps - 1)
        def _finish():
            for st, p in enumerate(partners):
                send_o[...] = acc_o[...].astype(jnp.bfloat16)
                rdma_o = pltpu.make_async_remote_copy(
                    src_ref=send_o, dst_ref=recv_o.at[st],
                    send_sem=send_sems.at[st, 0], recv_sem=recv_sems.at[st, 0],
                    device_id=p, device_id_type=pl.DeviceIdType.MESH,
                )
                rdma_l = pltpu.make_async_remote_copy(
                    src_ref=acc_l, dst_ref=recv_l.at[st],
                    send_sem=send_sems.at[st, 1], recv_sem=recv_sems.at[st, 1],
                    device_id=p, device_id_type=pl.DeviceIdType.MESH,
                )
                rdma_o.start()
                rdma_l.start()
                rdma_o.wait()
                rdma_l.wait()
                acc_o[...] += recv_o[st].astype(jnp.float32)
                acc_l[...] += recv_l[st]
            res = acc_o[...] / acc_l[...][:, :, None]
            out_ref[...] = jnp.transpose(res, (1, 0, 2)).reshape(B, 1, H, D)

    grid_spec = pltpu.PrefetchScalarGridSpec(
        num_scalar_prefetch=1,
        grid=(n_steps,),
        in_specs=[
            pl.BlockSpec((B, NB), lambda j, cb: (0, 0)),
            pl.BlockSpec((B, 1), lambda j, cb: (0, 0)),
            pl.BlockSpec((B, H * D), lambda j, cb: (0, 0)),
            pl.BlockSpec((CHUNK, BS, H * D), lambda j, cb: (cb[0] + j, 0, 0)),
            pl.BlockSpec((CHUNK, BS, H * D), lambda j, cb: (cb[0] + j, 0, 0)),
        ],
        out_specs=pl.BlockSpec((B, 1, H, D), lambda j, cb: (0, 0, 0, 0)),
        scratch_shapes=[
            pltpu.VMEM((H, B, D), jnp.float32),
            pltpu.VMEM((H, B), jnp.float32),
            pltpu.VMEM((H, B, D), jnp.bfloat16),
            pltpu.VMEM((3, H, B, D), jnp.bfloat16),
            pltpu.VMEM((3, H, B), jnp.float32),
            pltpu.SemaphoreType.DMA((3, 2)),
            pltpu.SemaphoreType.DMA((3, 2)),
        ],
    )

    return pl.pallas_call(
        body,
        grid_spec=grid_spec,
        out_shape=jax.ShapeDtypeStruct((B, 1, H, D), jnp.float32),
        compiler_params=pltpu.CompilerParams(
            dimension_semantics=("arbitrary",),
            collective_id=0,
            vmem_limit_bytes=100 * 1024 * 1024,
        ),
    )(chunk_base, bt, lens2, Q2, K2, V2)


# device time: 32163 ns/iter; 5.5932x vs baseline; 5.3931x over previous
import jax
import jax.numpy as jnp
from jax import lax
from jax.experimental import pallas as pl
from jax.experimental.pallas import tpu as pltpu

B = 32
H = 16
D = 128
BS = 32
NB = 256
QUAD = 4
CHUNK = 16
SCALE = D ** -0.5


def kernel(Q, K, V, bt, lens):
    n_pages = K.shape[0]
    my_pages = n_pages // QUAD
    n_steps = my_pages // CHUNK
    nkeys = CHUNK * BS
    lens2 = lens.reshape(B, 1)

    x = lax.axis_index("x")
    z = lax.axis_index("z")
    chunk_base = ((x * 2 + z) * n_steps).astype(jnp.int32).reshape(1)

    def body(cb_ref, bt_ref, lens_ref, q_ref, k_ref, v_ref, out_ref,
             acc_o, acc_l, send_o, recv_o, recv_l, send_sems, recv_sems):
        j = pl.program_id(0)
        my_x = lax.axis_index("x")
        my_y = lax.axis_index("y")
        my_z = lax.axis_index("z")
        partners = [
            (my_x, 1 - my_y, my_z),
            (1 - my_x, my_y, my_z),
            (my_x, my_y, 1 - my_z),
        ]

        @pl.when(j == 0)
        def _init():
            acc_o[...] = jnp.zeros_like(acc_o)
            acc_l[...] = jnp.zeros_like(acc_l)
            barrier_sem = pltpu.get_barrier_semaphore()
            for p in partners:
                pl.semaphore_signal(
                    barrier_sem, inc=1, device_id=p,
                    device_id_type=pl.DeviceIdType.MESH,
                )
            pl.semaphore_wait(barrier_sem, 3)

        page_ids = (my_y * n_pages + (cb_ref[0] + j) * CHUNK
                    + lax.broadcasted_iota(jnp.int32, (1, CHUNK, 1), 1))
        btv = bt_ref[...]
        col = lax.broadcasted_iota(jnp.int32, (1, 1, NB), 2)
        hit = (btv[:, None, :] == page_ids) & (col < lens_ref[...][:, :, None])
        counts = jnp.sum(hit.astype(jnp.float32), axis=2)
        counts_k = jnp.broadcast_to(
            counts[:, :, None], (B, CHUNK, BS)).reshape(B, nkeys)

        qt = jnp.transpose(
            (q_ref[...] * SCALE).reshape(B, H, D).astype(jnp.bfloat16),
            (1, 0, 2))
        kt = jnp.transpose(
            k_ref[...].reshape(nkeys, H, D).astype(jnp.bfloat16),
            (1, 0, 2))
        s = lax.dot_general(
            qt, kt, (((2,), (2,)), ((0,), (0,))),
            preferred_element_type=jnp.float32)
        e = jnp.exp(s) * counts_k[None, :, :]
        acc_l[...] += jnp.sum(e, axis=2)
        vt = jnp.transpose(
            v_ref[...].reshape(nkeys, H, D).astype(jnp.bfloat16),
            (1, 0, 2))
        acc_o[...] += lax.dot_general(
            e.astype(jnp.bfloat16), vt, (((2,), (1,)), ((0,), (0,))),
            preferred_element_type=jnp.float32)

        @pl.when(j == n_steps - 1)
        def _finish():
            for st, p in enumerate(partners):
                send_o[...] = acc_o[...].astype(jnp.bfloat16)
                rdma_o = pltpu.make_async_remote_copy(
                    src_ref=send_o, dst_ref=recv_o.at[st],
                    send_sem=send_sems.at[st, 0], recv_sem=recv_sems.at[st, 0],
                    device_id=p, device_id_type=pl.DeviceIdType.MESH,
                )
                rdma_l = pltpu.make_async_remote_copy(
                    src_ref=acc_l, dst_ref=recv_l.at[st],
                    send_sem=send_sems.at[st, 1], recv_sem=recv_sems.at[st, 1],
                    device_id=p, device_id_type=pl.DeviceIdType.MESH,
                )
                rdma_o.start()
                rdma_l.start()
                rdma_o.wait()
                rdma_l.wait()
                acc_o[...] += recv_o[st].astype(jnp.float32)
                acc_l[...] += recv_l[st]
            res = acc_o[...] / acc_l[...][:, :, None]
            out_ref[...] = jnp.transpose(res, (1, 0, 2)).reshape(B, 1, H, D)

    grid_spec = pltpu.PrefetchScalarGridSpec(
        num_scalar_prefetch=1,
        grid=(n_steps,),
        in_specs=[
            pl.BlockSpec((B, NB), lambda j, cb: (0, 0)),
            pl.BlockSpec((B, 1), lambda j, cb: (0, 0)),
            pl.BlockSpec((B, 1, H, D), lambda j, cb: (0, 0, 0, 0)),
            pl.BlockSpec((CHUNK, BS, H, D), lambda j, cb: (cb[0] + j, 0, 0, 0)),
            pl.BlockSpec((CHUNK, BS, H, D), lambda j, cb: (cb[0] + j, 0, 0, 0)),
        ],
        out_specs=pl.BlockSpec((B, 1, H, D), lambda j, cb: (0, 0, 0, 0)),
        scratch_shapes=[
            pltpu.VMEM((H, B, D), jnp.float32),
            pltpu.VMEM((H, B), jnp.float32),
            pltpu.VMEM((H, B, D), jnp.bfloat16),
            pltpu.VMEM((3, H, B, D), jnp.bfloat16),
            pltpu.VMEM((3, H, B), jnp.float32),
            pltpu.SemaphoreType.DMA((3, 2)),
            pltpu.SemaphoreType.DMA((3, 2)),
        ],
    )

    return pl.pallas_call(
        body,
        grid_spec=grid_spec,
        out_shape=jax.ShapeDtypeStruct((B, 1, H, D), jnp.float32),
        compiler_params=pltpu.CompilerParams(
            dimension_semantics=("arbitrary",),
            collective_id=0,
            vmem_limit_bytes=100 * 1024 * 1024,
        ),
    )(chunk_base, bt, lens2, Q, K, V)


# device time: 30770 ns/iter; 5.8464x vs baseline; 1.0453x over previous
import jax
import jax.numpy as jnp
from jax import lax
from jax.experimental import pallas as pl
from jax.experimental.pallas import tpu as pltpu

B = 32
H = 16
D = 128
BS = 32
NB = 256
QUAD = 4
CHUNK = 16
SCALE = D ** -0.5


def kernel(Q, K, V, bt, lens):
    n_pages = K.shape[0]
    my_pages = n_pages // QUAD
    n_steps = my_pages // CHUNK
    nkeys = CHUNK * BS
    lens2 = lens.reshape(B, 1)

    x = lax.axis_index("x")
    z = lax.axis_index("z")
    chunk_base = ((x * 2 + z) * n_steps).astype(jnp.int32).reshape(1)

    def body(cb_ref, bt_ref, lens_ref, q_ref, k_ref, v_ref, out_ref,
             acc_o, acc_l, send_o, send_o2, recv_o, recv_l,
             send_sems, recv_sems):
        j = pl.program_id(0)
        my_x = lax.axis_index("x")
        my_y = lax.axis_index("y")
        my_z = lax.axis_index("z")
        y_partner = (my_x, 1 - my_y, my_z)
        phase2 = [
            (1 - my_x, my_y, my_z),
            (my_x, my_y, 1 - my_z),
            (1 - my_x, my_y, 1 - my_z),
        ]
        partners = [y_partner] + phase2

        @pl.when(j == 0)
        def _init():
            acc_o[...] = jnp.zeros_like(acc_o)
            acc_l[...] = jnp.zeros_like(acc_l)
            barrier_sem = pltpu.get_barrier_semaphore()
            for p in partners:
                pl.semaphore_signal(
                    barrier_sem, inc=1, device_id=p,
                    device_id_type=pl.DeviceIdType.MESH,
                )
            pl.semaphore_wait(barrier_sem, len(partners))

        page_ids = (my_y * n_pages + (cb_ref[0] + j) * CHUNK
                    + lax.broadcasted_iota(jnp.int32, (1, CHUNK, 1), 1))
        btv = bt_ref[...]
        col = lax.broadcasted_iota(jnp.int32, (1, 1, NB), 2)
        hit = (btv[:, None, :] == page_ids) & (col < lens_ref[...][:, :, None])
        counts = jnp.sum(hit.astype(jnp.float32), axis=2)
        counts_k = jnp.broadcast_to(
            counts[:, :, None], (B, CHUNK, BS)).reshape(B, nkeys)

        qt = jnp.transpose(
            (q_ref[...] * SCALE).reshape(B, H, D).astype(jnp.bfloat16),
            (1, 0, 2))
        kt = jnp.transpose(
            k_ref[...].reshape(nkeys, H, D).astype(jnp.bfloat16),
            (1, 0, 2))
        s = lax.dot_general(
            qt, kt, (((2,), (2,)), ((0,), (0,))),
            preferred_element_type=jnp.float32)
        e = jnp.exp(s) * counts_k[None, :, :]
        acc_l[...] += jnp.sum(e, axis=2)
        vt = jnp.transpose(
            v_ref[...].reshape(nkeys, H, D).astype(jnp.bfloat16),
            (1, 0, 2))
        acc_o[...] += lax.dot_general(
            e.astype(jnp.bfloat16), vt, (((2,), (1,)), ((0,), (0,))),
            preferred_element_type=jnp.float32)

        @pl.when(j == n_steps - 1)
        def _finish():
            def exchange(st, p, src_o):
                rdma_o = pltpu.make_async_remote_copy(
                    src_ref=src_o, dst_ref=recv_o.at[st],
                    send_sem=send_sems.at[st, 0], recv_sem=recv_sems.at[st, 0],
                    device_id=p, device_id_type=pl.DeviceIdType.MESH,
                )
                rdma_l = pltpu.make_async_remote_copy(
                    src_ref=acc_l, dst_ref=recv_l.at[st],
                    send_sem=send_sems.at[st, 1], recv_sem=recv_sems.at[st, 1],
                    device_id=p, device_id_type=pl.DeviceIdType.MESH,
                )
                rdma_o.start()
                rdma_l.start()
                return rdma_o, rdma_l

            send_o[...] = acc_o[...].astype(jnp.bfloat16)
            rdma_o, rdma_l = exchange(0, y_partner, send_o)
            rdma_o.wait()
            rdma_l.wait()
            acc_o[...] += recv_o[0].astype(jnp.float32)
            acc_l[...] += recv_l[0]

            send_o2[...] = acc_o[...].astype(jnp.bfloat16)
            rdmas = [exchange(1 + i, p, send_o2)
                     for i, p in enumerate(phase2)]
            for rdma_o, rdma_l in rdmas:
                rdma_o.wait()
                rdma_l.wait()
            acc_o[...] += (recv_o[1].astype(jnp.float32)
                           + recv_o[2].astype(jnp.float32)
                           + recv_o[3].astype(jnp.float32))
            acc_l[...] += recv_l[1] + recv_l[2] + recv_l[3]

            res = acc_o[...] / acc_l[...][:, :, None]
            out_ref[...] = jnp.transpose(res, (1, 0, 2)).reshape(B, 1, H, D)

    grid_spec = pltpu.PrefetchScalarGridSpec(
        num_scalar_prefetch=1,
        grid=(n_steps,),
        in_specs=[
            pl.BlockSpec((B, NB), lambda j, cb: (0, 0)),
            pl.BlockSpec((B, 1), lambda j, cb: (0, 0)),
            pl.BlockSpec((B, 1, H, D), lambda j, cb: (0, 0, 0, 0)),
            pl.BlockSpec((CHUNK, BS, H, D), lambda j, cb: (cb[0] + j, 0, 0, 0)),
            pl.BlockSpec((CHUNK, BS, H, D), lambda j, cb: (cb[0] + j, 0, 0, 0)),
        ],
        out_specs=pl.BlockSpec((B, 1, H, D), lambda j, cb: (0, 0, 0, 0)),
        scratch_shapes=[
            pltpu.VMEM((H, B, D), jnp.float32),
            pltpu.VMEM((H, B), jnp.float32),
            pltpu.VMEM((H, B, D), jnp.bfloat16),
            pltpu.VMEM((H, B, D), jnp.bfloat16),
            pltpu.VMEM((4, H, B, D), jnp.bfloat16),
            pltpu.VMEM((4, H, B), jnp.float32),
            pltpu.SemaphoreType.DMA((4, 2)),
            pltpu.SemaphoreType.DMA((4, 2)),
        ],
    )

    return pl.pallas_call(
        body,
        grid_spec=grid_spec,
        out_shape=jax.ShapeDtypeStruct((B, 1, H, D), jnp.float32),
        compiler_params=pltpu.CompilerParams(
            dimension_semantics=("arbitrary",),
            collective_id=0,
            vmem_limit_bytes=100 * 1024 * 1024,
        ),
    )(chunk_base, bt, lens2, Q, K, V)
